# Initial kernel scaffold; baseline (speedup 1.0000x reference)
import jax
import jax.numpy as jnp
from jax.experimental import pallas as pl


def kernel(hidden_states, position_index, behavior_index, action_index, behavior_emb, Wg, Wu, Wd):
    raise NotImplementedError("write your pallas kernel here")


# speedup vs baseline: 1.8708x; 1.8708x over previous
import functools

import jax
import jax.numpy as jnp
from jax import lax
from jax.experimental import pallas as pl
from jax.experimental.pallas import tpu as pltpu
from jax.experimental.pallas import tpu_sc as plsc

NUM_EXPERTS = 8
TOTAL_EXPERTS = 8
HIDDEN = 2048
BEH_DIM = 64
INTER = 768
T = 2048

BM = 128
NBLK = T // BM
NSTEPS = NBLK + TOTAL_EXPERTS - 1

BEH_PAD = 128

NW = 32
ROWS_PER_W = T // NW
CH = 32
NCH = ROWS_PER_W // CH


def _route_meta(action_index, position_index):
    idx = jnp.maximum(
        (NUM_EXPERTS - 1) * (action_index.astype(jnp.int32) - 1)
        + position_index.astype(jnp.int32), 0)
    perm = jnp.argsort(idx).astype(jnp.int32)
    counts = jnp.bincount(idx, length=TOTAL_EXPERTS).astype(jnp.int32)
    ends = jnp.cumsum(counts)
    starts = ends - counts
    bfirst = starts // BM
    bcnt = jnp.where(counts > 0, (ends + BM - 1) // BM - bfirst, 0)
    co = jnp.cumsum(bcnt)
    s_ids = jnp.arange(NSTEPS, dtype=jnp.int32)
    e_s = jnp.searchsorted(co, s_ids, side="right").astype(jnp.int32)
    total = co[TOTAL_EXPERTS - 1]
    valid = s_ids < total
    e_c = jnp.minimum(e_s, TOTAL_EXPERTS - 1)
    prev = jnp.where(e_c > 0, co[jnp.maximum(e_c - 1, 0)], 0)
    r_s = bfirst[e_c] + (s_ids - prev)
    last = jnp.maximum(total - 1, 0)
    e_last = jnp.minimum(
        jnp.searchsorted(co, last, side="right").astype(jnp.int32),
        TOTAL_EXPERTS - 1)
    prev_last = jnp.where(e_last > 0, co[jnp.maximum(e_last - 1, 0)], 0)
    r_last = bfirst[e_last] + (last - prev_last)
    step_e = jnp.where(valid, e_c, e_last)
    step_r = jnp.where(valid, r_s, r_last)
    step_lo = jnp.where(valid, starts[e_c], 0)
    step_hi = jnp.where(valid, ends[e_c], 0)
    return idx, perm, step_e, step_r, step_lo, step_hi


def _moe_tc_body(se_ref, sr_ref, lo_ref, hi_ref,
                 xh_ref, xb_ref, wgh_ref, wgb_ref, wuh_ref, wub_ref, wd_ref,
                 out_ref):
    s = pl.program_id(0)
    lo = lo_ref[s]
    hi = hi_ref[s]
    r = sr_ref[s]

    @pl.when(hi > lo)
    def _():
        xh = xh_ref[...]
        xb = xb_ref[...][:, :BEH_DIM]
        g = (jnp.dot(xh, wgh_ref[0], preferred_element_type=jnp.float32)
             + jnp.dot(xb, wgb_ref[0], preferred_element_type=jnp.float32))
        u = (jnp.dot(xh, wuh_ref[0], preferred_element_type=jnp.float32)
             + jnp.dot(xb, wub_ref[0], preferred_element_type=jnp.float32))
        h = g * jax.nn.sigmoid(g) * u
        y = jnp.dot(h, wd_ref[0], preferred_element_type=jnp.float32)
        gid = r * BM + lax.broadcasted_iota(jnp.int32, (BM, 1), 0)
        m = (gid >= lo) & (gid < hi)
        out_ref[...] = jnp.where(m, y, out_ref[...])


def _tc_moe(step_e, step_r, step_lo, step_hi, xh_s, xb_s, Wg, Wu, Wd):
    grid_spec = pltpu.PrefetchScalarGridSpec(
        num_scalar_prefetch=4,
        grid=(NSTEPS,),
        in_specs=[
            pl.BlockSpec((BM, HIDDEN), lambda s, se, sr, lo, hi: (sr[s], 0)),
            pl.BlockSpec((BM, BEH_PAD), lambda s, se, sr, lo, hi: (sr[s], 0)),
            pl.BlockSpec((1, HIDDEN, INTER),
                         lambda s, se, sr, lo, hi: (se[s], 0, 0)),
            pl.BlockSpec((1, BEH_DIM, INTER),
                         lambda s, se, sr, lo, hi: (se[s], HIDDEN // BEH_DIM, 0)),
            pl.BlockSpec((1, HIDDEN, INTER),
                         lambda s, se, sr, lo, hi: (se[s], 0, 0)),
            pl.BlockSpec((1, BEH_DIM, INTER),
                         lambda s, se, sr, lo, hi: (se[s], HIDDEN // BEH_DIM, 0)),
            pl.BlockSpec((1, INTER, HIDDEN),
                         lambda s, se, sr, lo, hi: (se[s], 0, 0)),
        ],
        out_specs=pl.BlockSpec((BM, HIDDEN), lambda s, se, sr, lo, hi: (sr[s], 0)),
    )
    return pl.pallas_call(
        _moe_tc_body,
        grid_spec=grid_spec,
        out_shape=jax.ShapeDtypeStruct((T, HIDDEN), jnp.float32),
        compiler_params=pltpu.CompilerParams(
            dimension_semantics=("arbitrary",)),
    )(step_e, step_r, step_lo, step_hi, xh_s, xb_s, Wg, Wg, Wu, Wu, Wd)


def _sc_gather(hidden_states, behavior_emb, perm, bidx_sorted):
    mesh = plsc.VectorSubcoreMesh(core_axis_name="c", subcore_axis_name="s")

    @functools.partial(
        pl.kernel, mesh=mesh,
        out_type=[jax.ShapeDtypeStruct((T, HIDDEN), jnp.float32),
                  jax.ShapeDtypeStruct((T, BEH_PAD), jnp.float32)],
        scratch_types=[pltpu.VMEM((CH,), jnp.int32),
                       pltpu.VMEM((CH,), jnp.int32),
                       pltpu.VMEM((CH, HIDDEN), jnp.float32),
                       pltpu.VMEM((CH, BEH_PAD), jnp.float32),
                       pltpu.SemaphoreType.DMA,
                       pltpu.SemaphoreType.DMA],
    )
    def gather_k(hid_hbm, bemb_hbm, perm_hbm, bidx_hbm, xh_hbm, xb_hbm,
                 idx_v, bidx_v, hrow_v, brow_v, sem1, sem2):
        wid = lax.axis_index("s") * 2 + lax.axis_index("c")
        for c in range(NCH):
            base = wid * ROWS_PER_W + c * CH
            pltpu.sync_copy(perm_hbm.at[pl.ds(base, CH)], idx_v)
            pltpu.sync_copy(bidx_hbm.at[pl.ds(base, CH)], bidx_v)
            cp1 = pltpu.async_copy(hid_hbm.at[idx_v], hrow_v, sem1)
            cp2 = pltpu.async_copy(bemb_hbm.at[bidx_v], brow_v, sem2)
            cp1.wait()
            cp2.wait()
            pltpu.sync_copy(hrow_v, xh_hbm.at[pl.ds(base, CH)])
            pltpu.sync_copy(brow_v, xb_hbm.at[pl.ds(base, CH)])

    return gather_k(hidden_states, behavior_emb, perm, bidx_sorted)


def _sc_scatter(y_sorted, perm):
    mesh = plsc.VectorSubcoreMesh(core_axis_name="c", subcore_axis_name="s")

    @functools.partial(
        pl.kernel, mesh=mesh,
        out_type=jax.ShapeDtypeStruct((T, HIDDEN), jnp.float32),
        scratch_types=[pltpu.VMEM((CH,), jnp.int32),
                       pltpu.VMEM((CH, HIDDEN), jnp.float32),
                       pltpu.SemaphoreType.DMA],
    )
    def scatter_k(y_hbm, perm_hbm, out_hbm, idx_v, row_v, sem):
        wid = lax.axis_index("s") * 2 + lax.axis_index("c")
        for c in range(NCH):
            base = wid * ROWS_PER_W + c * CH
            pltpu.sync_copy(perm_hbm.at[pl.ds(base, CH)], idx_v)
            pltpu.sync_copy(y_hbm.at[pl.ds(base, CH)], row_v)
            pltpu.async_copy(row_v, out_hbm.at[idx_v], sem).wait()

    return scatter_k(y_sorted, perm)


def kernel(hidden_states, position_index, behavior_index, action_index,
           behavior_emb, Wg, Wu, Wd):
    _, perm, step_e, step_r, step_lo, step_hi = _route_meta(
        action_index, position_index)
    bidx_sorted = behavior_index.astype(jnp.int32)[perm]
    bemb_pad = jnp.pad(behavior_emb, ((0, 0), (0, BEH_PAD - BEH_DIM)))
    xh_s, xb_s = _sc_gather(hidden_states, bemb_pad, perm, bidx_sorted)
    y_s = _tc_moe(step_e, step_r, step_lo, step_hi, xh_s, xb_s, Wg, Wu, Wd)
    return _sc_scatter(y_s, perm)


# speedup vs baseline: 1.8772x; 1.0034x over previous
import functools

import jax
import jax.numpy as jnp
from jax import lax
from jax.experimental import pallas as pl
from jax.experimental.pallas import tpu as pltpu
from jax.experimental.pallas import tpu_sc as plsc

NUM_EXPERTS = 8
TOTAL_EXPERTS = 8
HIDDEN = 2048
BEH_DIM = 64
INTER = 768
T = 2048

BM = 128
NBLK = T // BM
NSTEPS = NBLK + TOTAL_EXPERTS - 1

BEH_PAD = 128

NW = 32
ROWS_PER_W = T // NW
CH = 32
NCH = ROWS_PER_W // CH


def _route_meta(action_index, position_index):
    idx = jnp.maximum(
        (NUM_EXPERTS - 1) * (action_index.astype(jnp.int32) - 1)
        + position_index.astype(jnp.int32), 0)
    perm = jnp.argsort(idx).astype(jnp.int32)
    counts = jnp.bincount(idx, length=TOTAL_EXPERTS).astype(jnp.int32)
    ends = jnp.cumsum(counts)
    starts = ends - counts
    bfirst = starts // BM
    bcnt = jnp.where(counts > 0, (ends + BM - 1) // BM - bfirst, 0)
    co = jnp.cumsum(bcnt)
    s_ids = jnp.arange(NSTEPS, dtype=jnp.int32)
    e_s = jnp.searchsorted(co, s_ids, side="right").astype(jnp.int32)
    total = co[TOTAL_EXPERTS - 1]
    valid = s_ids < total
    e_c = jnp.minimum(e_s, TOTAL_EXPERTS - 1)
    prev = jnp.where(e_c > 0, co[jnp.maximum(e_c - 1, 0)], 0)
    r_s = bfirst[e_c] + (s_ids - prev)
    last = jnp.maximum(total - 1, 0)
    e_last = jnp.minimum(
        jnp.searchsorted(co, last, side="right").astype(jnp.int32),
        TOTAL_EXPERTS - 1)
    prev_last = jnp.where(e_last > 0, co[jnp.maximum(e_last - 1, 0)], 0)
    r_last = bfirst[e_last] + (last - prev_last)
    step_e = jnp.where(valid, e_c, e_last)
    step_r = jnp.where(valid, r_s, r_last)
    step_lo = jnp.where(valid, starts[e_c], 0)
    step_hi = jnp.where(valid, ends[e_c], 0)
    return idx, perm, step_e, step_r, step_lo, step_hi


def _moe_tc_body(se_ref, sr_ref, lo_ref, hi_ref,
                 xh_ref, xb_ref, wgh_ref, wgb_ref, wuh_ref, wub_ref, wd_ref,
                 out_ref):
    s = pl.program_id(0)
    lo = lo_ref[s]
    hi = hi_ref[s]
    r = sr_ref[s]

    @pl.when(hi > lo)
    def _():
        bf = jnp.bfloat16
        xh = xh_ref[...].astype(bf)
        xb = xb_ref[...][:, :BEH_DIM].astype(bf)
        g = (jnp.dot(xh, wgh_ref[0].astype(bf), preferred_element_type=jnp.float32)
             + jnp.dot(xb, wgb_ref[0].astype(bf), preferred_element_type=jnp.float32))
        u = (jnp.dot(xh, wuh_ref[0].astype(bf), preferred_element_type=jnp.float32)
             + jnp.dot(xb, wub_ref[0].astype(bf), preferred_element_type=jnp.float32))
        h = (g * jax.nn.sigmoid(g) * u).astype(bf)
        y = jnp.dot(h, wd_ref[0].astype(bf), preferred_element_type=jnp.float32)
        gid = r * BM + lax.broadcasted_iota(jnp.int32, (BM, 1), 0)
        m = (gid >= lo) & (gid < hi)
        out_ref[...] = jnp.where(m, y, out_ref[...])


def _tc_moe(step_e, step_r, step_lo, step_hi, xh_s, xb_s, Wg, Wu, Wd):
    grid_spec = pltpu.PrefetchScalarGridSpec(
        num_scalar_prefetch=4,
        grid=(NSTEPS,),
        in_specs=[
            pl.BlockSpec((BM, HIDDEN), lambda s, se, sr, lo, hi: (sr[s], 0)),
            pl.BlockSpec((BM, BEH_PAD), lambda s, se, sr, lo, hi: (sr[s], 0)),
            pl.BlockSpec((1, HIDDEN, INTER),
                         lambda s, se, sr, lo, hi: (se[s], 0, 0)),
            pl.BlockSpec((1, BEH_DIM, INTER),
                         lambda s, se, sr, lo, hi: (se[s], HIDDEN // BEH_DIM, 0)),
            pl.BlockSpec((1, HIDDEN, INTER),
                         lambda s, se, sr, lo, hi: (se[s], 0, 0)),
            pl.BlockSpec((1, BEH_DIM, INTER),
                         lambda s, se, sr, lo, hi: (se[s], HIDDEN // BEH_DIM, 0)),
            pl.BlockSpec((1, INTER, HIDDEN),
                         lambda s, se, sr, lo, hi: (se[s], 0, 0)),
        ],
        out_specs=pl.BlockSpec((BM, HIDDEN), lambda s, se, sr, lo, hi: (sr[s], 0)),
    )
    return pl.pallas_call(
        _moe_tc_body,
        grid_spec=grid_spec,
        out_shape=jax.ShapeDtypeStruct((T, HIDDEN), jnp.float32),
        compiler_params=pltpu.CompilerParams(
            dimension_semantics=("arbitrary",)),
    )(step_e, step_r, step_lo, step_hi, xh_s, xb_s, Wg, Wg, Wu, Wu, Wd)


def _sc_gather(hidden_states, behavior_emb, perm, bidx_sorted):
    mesh = plsc.VectorSubcoreMesh(core_axis_name="c", subcore_axis_name="s")

    @functools.partial(
        pl.kernel, mesh=mesh,
        out_type=[jax.ShapeDtypeStruct((T, HIDDEN), jnp.float32),
                  jax.ShapeDtypeStruct((T, BEH_PAD), jnp.float32)],
        scratch_types=[pltpu.VMEM((CH,), jnp.int32),
                       pltpu.VMEM((CH,), jnp.int32),
                       pltpu.VMEM((CH, HIDDEN), jnp.float32),
                       pltpu.VMEM((CH, BEH_PAD), jnp.float32),
                       pltpu.SemaphoreType.DMA,
                       pltpu.SemaphoreType.DMA],
    )
    def gather_k(hid_hbm, bemb_hbm, perm_hbm, bidx_hbm, xh_hbm, xb_hbm,
                 idx_v, bidx_v, hrow_v, brow_v, sem1, sem2):
        wid = lax.axis_index("s") * 2 + lax.axis_index("c")
        for c in range(NCH):
            base = wid * ROWS_PER_W + c * CH
            pltpu.sync_copy(perm_hbm.at[pl.ds(base, CH)], idx_v)
            pltpu.sync_copy(bidx_hbm.at[pl.ds(base, CH)], bidx_v)
            cp1 = pltpu.async_copy(hid_hbm.at[idx_v], hrow_v, sem1)
            cp2 = pltpu.async_copy(bemb_hbm.at[bidx_v], brow_v, sem2)
            cp1.wait()
            cp2.wait()
            pltpu.sync_copy(hrow_v, xh_hbm.at[pl.ds(base, CH)])
            pltpu.sync_copy(brow_v, xb_hbm.at[pl.ds(base, CH)])

    return gather_k(hidden_states, behavior_emb, perm, bidx_sorted)


def _sc_scatter(y_sorted, perm):
    mesh = plsc.VectorSubcoreMesh(core_axis_name="c", subcore_axis_name="s")

    @functools.partial(
        pl.kernel, mesh=mesh,
        out_type=jax.ShapeDtypeStruct((T, HIDDEN), jnp.float32),
        scratch_types=[pltpu.VMEM((CH,), jnp.int32),
                       pltpu.VMEM((CH, HIDDEN), jnp.float32),
                       pltpu.SemaphoreType.DMA],
    )
    def scatter_k(y_hbm, perm_hbm, out_hbm, idx_v, row_v, sem):
        wid = lax.axis_index("s") * 2 + lax.axis_index("c")
        for c in range(NCH):
            base = wid * ROWS_PER_W + c * CH
            pltpu.sync_copy(perm_hbm.at[pl.ds(base, CH)], idx_v)
            pltpu.sync_copy(y_hbm.at[pl.ds(base, CH)], row_v)
            pltpu.async_copy(row_v, out_hbm.at[idx_v], sem).wait()

    return scatter_k(y_sorted, perm)


def kernel(hidden_states, position_index, behavior_index, action_index,
           behavior_emb, Wg, Wu, Wd):
    _, perm, step_e, step_r, step_lo, step_hi = _route_meta(
        action_index, position_index)
    bidx_sorted = behavior_index.astype(jnp.int32)[perm]
    bemb_pad = jnp.pad(behavior_emb, ((0, 0), (0, BEH_PAD - BEH_DIM)))
    xh_s, xb_s = _sc_gather(hidden_states, bemb_pad, perm, bidx_sorted)
    y_s = _tc_moe(step_e, step_r, step_lo, step_hi, xh_s, xb_s, Wg, Wu, Wd)
    return _sc_scatter(y_s, perm)
